# baseline (speedup 1.0000x reference)
import functools

import jax
import jax.numpy as jnp
import numpy as np
from jax.experimental import pallas as pl
from jax.experimental.pallas import tpu as pltpu
from jax.experimental.pallas import tpu_sc as plsc


def _conv(x, w, b, stride=1, pad=1):
    y = jax.lax.conv_general_dilated(
        x, w, (stride, stride), [(pad, pad), (pad, pad)],
        dimension_numbers=('NCHW', 'OIHW', 'NCHW'))
    return y + b[None, :, None, None]


def _up2(x):
    return jnp.repeat(jnp.repeat(x, 2, axis=2), 2, axis=3)


_ROWS = 3136
_BLK = 448
_K = 1024
_D = 64


_NW = 32
_BPAD = 3328
_BPW = _BPAD // _NW


def _pq_table_kernel(cb_ref, w_ref, b_ref, o_ref):
    o_ref[...] = (jnp.dot(cb_ref[...], w_ref[...],
                          preferred_element_type=jnp.float32) + b_ref[...])


def _pq_table(cb, pqm, pqb):
    return pl.pallas_call(
        _pq_table_kernel,
        out_shape=jax.ShapeDtypeStruct((_K, 256), jnp.float32),
    )(cb, pqm, pqb[None, :])


def _loss_kernel(q_ref, z_ref, o_ref):
    d = q_ref[...] - z_ref[...]
    o_ref[...] = jnp.sum(d * d).reshape(1, 1)


def _commit_loss(q, zf):
    s = pl.pallas_call(
        _loss_kernel,
        out_shape=jax.ShapeDtypeStruct((1, 1), jnp.float32),
    )(q, zf)
    return s[0, 0] / (_ROWS * _D)


def _sc_gather(cb, cb2, idx):
    idx_pad = jnp.pad(idx, (0, _BPAD - idx.shape[0]))
    mesh = plsc.VectorSubcoreMesh(core_axis_name="c", subcore_axis_name="s")

    @functools.partial(
        pl.kernel, mesh=mesh,
        out_type=[jax.ShapeDtypeStruct((_BPAD, 128), jnp.float32),
                  jax.ShapeDtypeStruct((_BPAD, 256), jnp.float32)],
        scratch_types=[pltpu.VMEM((_BPW,), jnp.int32),
                       pltpu.VMEM((_BPW, 128), jnp.float32),
                       pltpu.VMEM((_BPW, 256), jnp.float32),
                       pltpu.SemaphoreType.DMA,
                       pltpu.SemaphoreType.DMA],
    )
    def k(cbp_hbm, cb2_hbm, idx_hbm, q_hbm, g_hbm, idx_v, q_v, g_v, s1, s2):
        wid = jax.lax.axis_index("s") * 2 + jax.lax.axis_index("c")
        base = wid * _BPW
        pltpu.sync_copy(idx_hbm.at[pl.ds(base, _BPW)], idx_v)
        c1 = pltpu.async_copy(cbp_hbm.at[idx_v], q_v, s1)
        c2 = pltpu.async_copy(cb2_hbm.at[idx_v], g_v, s2)
        c1.wait()
        c2.wait()
        pltpu.sync_copy(q_v, q_hbm.at[pl.ds(base, _BPW)])
        pltpu.sync_copy(g_v, g_hbm.at[pl.ds(base, _BPW)])

    cb_pad = jnp.pad(cb, ((0, 0), (0, 128 - _D)))
    q, g = k(cb_pad, cb2, idx_pad)
    return q[:_ROWS, :_D], g[:_ROWS]


def _phase_weights(dec_w):
    t = jnp.array([[[1, 0, 0], [0, 1, 1]],
                   [[1, 1, 0], [0, 0, 1]]], jnp.float32)
    w2 = jnp.einsum('aup,bvq,oipq->abuvio', t, t, dec_w)
    co, ci = dec_w.shape[0], dec_w.shape[1]
    return w2.reshape(4, 4, ci, co).astype(jnp.bfloat16)


def _make_upconv_kernel(H, Wp, relu, out_dtype, nchunks, pack_phases):
    L = H * Wp
    CH = L // nchunks

    def _k(x_ref, w_ref, b_ref, o_ref):
        for c in range(nchunks):
            accs = []
            for a in (0, 1):
                for b in (0, 1):
                    acc = jnp.zeros((CH, w_ref.shape[3]), jnp.float32)
                    for u in (0, 1):
                        for v in (0, 1):
                            s = (1 + a + u) * Wp + (b + v - 1) + c * CH
                            xs = x_ref[0, pl.ds(s, CH), :]
                            acc += jnp.dot(xs, w_ref[2 * a + b, 2 * u + v],
                                           preferred_element_type=jnp.float32)
                    acc = acc + b_ref[...]
                    if relu:
                        acc = jnp.maximum(acc, 0.0)
                    acc = acc.astype(out_dtype)
                    if pack_phases:
                        accs.append(acc)
                    else:
                        o_ref[0, 2 * a + b, pl.ds(c * CH, CH), :] = acc
            if pack_phases:
                o_ref[0, pl.ds(c * CH, CH), :] = jnp.concatenate(accs, axis=1)

    return _k


def _upconv(g, w, bias, relu, out_dtype):
    n, h, wdim, ci = g.shape
    co = w.shape[0]
    wp = wdim + 2
    w2 = _phase_weights(w)
    gp = jnp.pad(g.astype(jnp.bfloat16), ((0, 0), (2, 2), (1, 1), (0, 0)))
    gp = gp.reshape(n, (h + 4) * wp, ci)
    l = h * wp
    nchunks = 1 if l <= 4096 else 4
    pack = co < 128
    if pack:
        out_specs = pl.BlockSpec((1, l, 4 * co), lambda i: (i, 0, 0))
        out_shape = jax.ShapeDtypeStruct((n, l, 4 * co), out_dtype)
    else:
        out_specs = pl.BlockSpec((1, 4, l, co), lambda i: (i, 0, 0, 0))
        out_shape = jax.ShapeDtypeStruct((n, 4, l, co), out_dtype)
    out = pl.pallas_call(
        _make_upconv_kernel(h, wp, relu, out_dtype, nchunks, pack),
        grid=(n,),
        in_specs=[
            pl.BlockSpec((1, (h + 4) * wp, ci), lambda i: (i, 0, 0)),
            pl.BlockSpec((4, 4, ci, co), lambda i: (0, 0, 0, 0)),
            pl.BlockSpec((1, co), lambda i: (0, 0)),
        ],
        out_specs=out_specs,
        out_shape=out_shape,
    )(gp, w2, bias[None, :].astype(jnp.float32))
    if pack:
        out = out.reshape(n, h, wp, 2, 2, co)[:, :, 1:wdim + 1]
        out = jnp.transpose(out, (0, 1, 3, 2, 4, 5))
    else:
        out = out.reshape(n, 2, 2, h, wp, co)[:, :, :, :, 1:wdim + 1, :]
        out = jnp.transpose(out, (0, 3, 1, 4, 2, 5))
    return out.reshape(n, 2 * h, 2 * wdim, co)


def kernel(x, enc_w1, enc_b1, enc_w2, enc_b2, enc_w3, enc_b3, quant_w,
           quant_b, codebook, pq_w, pq_b, dec_w1, dec_b1, dec_w2, dec_b2,
           dec_w3, dec_b3):
    h = jax.nn.relu(_conv(x, enc_w1, enc_b1, stride=2))
    h = jax.nn.relu(_conv(h, enc_w2, enc_b2, stride=2))
    h = _conv(h, enc_w3, enc_b3, stride=2)

    z = _conv(h, quant_w, quant_b, stride=1, pad=0)
    z = jnp.transpose(z, (0, 2, 3, 1))
    zf = z.reshape(-1, 64)
    dd = (jnp.sum(zf * zf, axis=1, keepdims=True) - 2.0 * (zf @ codebook.T)
          + jnp.sum(codebook * codebook, axis=1)[None, :])
    idx = jnp.argmin(dd, axis=1).astype(jnp.int32)

    pqm = pq_w[:, :, 0, 0].T
    cb2 = _pq_table(codebook, pqm, pq_b)
    q, g_vec = _sc_gather(codebook, cb2, idx)
    commit_loss = _commit_loss(q, zf)

    g = g_vec.reshape(4, 28, 28, 256)
    g = _upconv(g, dec_w1, dec_b1, relu=True, out_dtype=jnp.bfloat16)
    g = _upconv(g, dec_w2, dec_b2, relu=True, out_dtype=jnp.bfloat16)
    g = _upconv(g, dec_w3, dec_b3, relu=False, out_dtype=jnp.float32)
    decoded = jnp.transpose(g, (0, 3, 1, 2))
    return (commit_loss, decoded)


# speedup vs baseline: 1.2443x; 1.2443x over previous
import jax
import jax.numpy as jnp
import numpy as np
from jax.experimental import pallas as pl
from jax.experimental.pallas import tpu as pltpu


def _conv(x, w, b, stride=1, pad=1):
    y = jax.lax.conv_general_dilated(
        x, w, (stride, stride), [(pad, pad), (pad, pad)],
        dimension_numbers=('NCHW', 'OIHW', 'NCHW'))
    return y + b[None, :, None, None]


def _up2(x):
    return jnp.repeat(jnp.repeat(x, 2, axis=2), 2, axis=3)


_ROWS = 3136
_BLK = 448
_K = 1024
_D = 64


def _pq_table_kernel(cb_ref, w_ref, b_ref, o_ref):
    o_ref[...] = (jnp.dot(cb_ref[...], w_ref[...],
                          preferred_element_type=jnp.float32)
                  + b_ref[...]).astype(jnp.bfloat16)


def _pq_table(cb, pqm, pqb):
    return pl.pallas_call(
        _pq_table_kernel,
        out_shape=jax.ShapeDtypeStruct((_K, 256), jnp.bfloat16),
    )(cb, pqm, pqb[None, :])


def _vq_post_kernel(idx_ref, cb_ref, cb2_ref, zf_ref, g_ref, ls_ref):
    i = pl.program_id(0)
    oh = (jax.lax.broadcasted_iota(jnp.int32, (_BLK, _K), 1)
          == idx_ref[...]).astype(jnp.float32)
    q = jnp.dot(oh, cb_ref[...], preferred_element_type=jnp.float32)
    dq = q - zf_ref[...]
    part = jnp.sum(dq * dq).reshape(1, 1)

    @pl.when(i == 0)
    def _init():
        ls_ref[...] = jnp.zeros_like(part)

    ls_ref[...] += part
    g_ref[...] = jnp.dot(oh.astype(jnp.bfloat16), cb2_ref[...],
                         preferred_element_type=jnp.float32
                         ).astype(jnp.bfloat16)


def _vq_post(idx, cb, cb2, zf):
    g, lsum = pl.pallas_call(
        _vq_post_kernel,
        grid=(_ROWS // _BLK,),
        in_specs=[
            pl.BlockSpec((_BLK, 1), lambda i: (i, 0)),
            pl.BlockSpec((_K, _D), lambda i: (0, 0)),
            pl.BlockSpec((_K, 256), lambda i: (0, 0)),
            pl.BlockSpec((_BLK, _D), lambda i: (i, 0)),
        ],
        out_specs=[
            pl.BlockSpec((_BLK, 256), lambda i: (i, 0)),
            pl.BlockSpec((1, 1), lambda i: (0, 0)),
        ],
        out_shape=[
            jax.ShapeDtypeStruct((_ROWS, 256), jnp.bfloat16),
            jax.ShapeDtypeStruct((1, 1), jnp.float32),
        ],
    )(idx[:, None], cb, cb2, zf)
    return g, lsum[0, 0] / (_ROWS * _D)


def _phase_weights(dec_w):
    t = jnp.array([[[1, 0, 0], [0, 1, 1]],
                   [[1, 1, 0], [0, 0, 1]]], jnp.float32)
    w2 = jnp.einsum('aup,bvq,oipq->abuvio', t, t, dec_w)
    co, ci = dec_w.shape[0], dec_w.shape[1]
    return w2.reshape(4, 4, ci, co).astype(jnp.bfloat16)


def _make_upconv_kernel(H, Wp, relu, out_dtype, nchunks, pack_phases):
    L = H * Wp
    CH = L // nchunks

    def _k(x_ref, w_ref, b_ref, o_ref):
        for c in range(nchunks):
            accs = []
            for a in (0, 1):
                for b in (0, 1):
                    acc = jnp.zeros((CH, w_ref.shape[3]), jnp.float32)
                    for u in (0, 1):
                        for v in (0, 1):
                            s = (1 + a + u) * Wp + (b + v - 1) + c * CH
                            xs = x_ref[0, pl.ds(s, CH), :]
                            acc += jnp.dot(xs, w_ref[2 * a + b, 2 * u + v],
                                           preferred_element_type=jnp.float32)
                    acc = acc + b_ref[...]
                    if relu:
                        acc = jnp.maximum(acc, 0.0)
                    acc = acc.astype(out_dtype)
                    if pack_phases:
                        accs.append(acc)
                    else:
                        o_ref[0, 2 * a + b, pl.ds(c * CH, CH), :] = acc
            if pack_phases:
                o_ref[0, pl.ds(c * CH, CH), :] = jnp.concatenate(accs, axis=1)

    return _k


def _upconv(g, w, bias, relu, out_dtype):
    n, h, wdim, ci = g.shape
    co = w.shape[0]
    wp = wdim + 2
    w2 = _phase_weights(w)
    gp = jnp.pad(g.astype(jnp.bfloat16), ((0, 0), (2, 2), (1, 1), (0, 0)))
    gp = gp.reshape(n, (h + 4) * wp, ci)
    l = h * wp
    nchunks = 1 if l <= 4096 else 4
    pack = co < 128
    if pack:
        out_specs = pl.BlockSpec((1, l, 4 * co), lambda i: (i, 0, 0))
        out_shape = jax.ShapeDtypeStruct((n, l, 4 * co), out_dtype)
    else:
        out_specs = pl.BlockSpec((1, 4, l, co), lambda i: (i, 0, 0, 0))
        out_shape = jax.ShapeDtypeStruct((n, 4, l, co), out_dtype)
    out = pl.pallas_call(
        _make_upconv_kernel(h, wp, relu, out_dtype, nchunks, pack),
        grid=(n,),
        in_specs=[
            pl.BlockSpec((1, (h + 4) * wp, ci), lambda i: (i, 0, 0)),
            pl.BlockSpec((4, 4, ci, co), lambda i: (0, 0, 0, 0)),
            pl.BlockSpec((1, co), lambda i: (0, 0)),
        ],
        out_specs=out_specs,
        out_shape=out_shape,
    )(gp, w2, bias[None, :].astype(jnp.float32))
    if pack:
        out = out.reshape(n, h, wp, 2, 2, co)[:, :, 1:wdim + 1]
        out = jnp.transpose(out, (0, 1, 3, 2, 4, 5))
    else:
        out = out.reshape(n, 2, 2, h, wp, co)[:, :, :, :, 1:wdim + 1, :]
        out = jnp.transpose(out, (0, 3, 1, 4, 2, 5))
    return out.reshape(n, 2 * h, 2 * wdim, co)


def kernel(x, enc_w1, enc_b1, enc_w2, enc_b2, enc_w3, enc_b3, quant_w,
           quant_b, codebook, pq_w, pq_b, dec_w1, dec_b1, dec_w2, dec_b2,
           dec_w3, dec_b3):
    h = jax.nn.relu(_conv(x, enc_w1, enc_b1, stride=2))
    h = jax.nn.relu(_conv(h, enc_w2, enc_b2, stride=2))
    h = _conv(h, enc_w3, enc_b3, stride=2)

    z = _conv(h, quant_w, quant_b, stride=1, pad=0)
    z = jnp.transpose(z, (0, 2, 3, 1))
    zf = z.reshape(-1, 64)
    dd = (jnp.sum(zf * zf, axis=1, keepdims=True) - 2.0 * (zf @ codebook.T)
          + jnp.sum(codebook * codebook, axis=1)[None, :])
    idx = jnp.argmin(dd, axis=1).astype(jnp.int32)

    pqm = pq_w[:, :, 0, 0].T
    cb2 = _pq_table(codebook, pqm, pq_b)
    g_vec, commit_loss = _vq_post(idx, codebook, cb2, zf)

    g = g_vec.reshape(4, 28, 28, 256)
    g = _upconv(g, dec_w1, dec_b1, relu=True, out_dtype=jnp.bfloat16)
    g = _upconv(g, dec_w2, dec_b2, relu=True, out_dtype=jnp.bfloat16)
    g = _upconv(g, dec_w3, dec_b3, relu=False, out_dtype=jnp.float32)
    decoded = jnp.transpose(g, (0, 3, 1, 2))
    return (commit_loss, decoded)


# speedup vs baseline: 1.3937x; 1.1201x over previous
import jax
import jax.numpy as jnp
import numpy as np
from jax.experimental import pallas as pl
from jax.experimental.pallas import tpu as pltpu


def _conv(x, w, b, stride=1, pad=1):
    y = jax.lax.conv_general_dilated(
        x, w, (stride, stride), [(pad, pad), (pad, pad)],
        dimension_numbers=('NCHW', 'OIHW', 'NCHW'))
    return y + b[None, :, None, None]


def _up2(x):
    return jnp.repeat(jnp.repeat(x, 2, axis=2), 2, axis=3)


_ROWS = 3136
_BLK = 448
_K = 1024
_D = 64


def _pq_table_kernel(cb_ref, w_ref, b_ref, o_ref):
    o_ref[...] = (jnp.dot(cb_ref[...], w_ref[...],
                          preferred_element_type=jnp.float32)
                  + b_ref[...]).astype(jnp.bfloat16)


def _pq_table(cb, pqm, pqb):
    return pl.pallas_call(
        _pq_table_kernel,
        out_shape=jax.ShapeDtypeStruct((_K, 256), jnp.bfloat16),
    )(cb, pqm, pqb[None, :])


def _vq_post_kernel(idx_ref, cb_ref, cb2_ref, zf_ref, g_ref, ls_ref):
    i = pl.program_id(0)
    oh = (jax.lax.broadcasted_iota(jnp.int32, (_BLK, _K), 1)
          == idx_ref[...]).astype(jnp.float32)
    q = jnp.dot(oh, cb_ref[...], preferred_element_type=jnp.float32)
    dq = q - zf_ref[...]
    part = jnp.sum(dq * dq).reshape(1, 1)

    @pl.when(i == 0)
    def _init():
        ls_ref[...] = jnp.zeros_like(part)

    ls_ref[...] += part
    g_ref[...] = jnp.dot(oh.astype(jnp.bfloat16), cb2_ref[...],
                         preferred_element_type=jnp.float32
                         ).astype(jnp.bfloat16)


def _vq_post(idx, cb, cb2, zf):
    g, lsum = pl.pallas_call(
        _vq_post_kernel,
        grid=(_ROWS // _BLK,),
        in_specs=[
            pl.BlockSpec((_BLK, 1), lambda i: (i, 0)),
            pl.BlockSpec((_K, _D), lambda i: (0, 0)),
            pl.BlockSpec((_K, 256), lambda i: (0, 0)),
            pl.BlockSpec((_BLK, _D), lambda i: (i, 0)),
        ],
        out_specs=[
            pl.BlockSpec((_BLK, 256), lambda i: (i, 0)),
            pl.BlockSpec((1, 1), lambda i: (0, 0)),
        ],
        out_shape=[
            jax.ShapeDtypeStruct((_ROWS, 256), jnp.bfloat16),
            jax.ShapeDtypeStruct((1, 1), jnp.float32),
        ],
    )(idx[:, None], cb, cb2, zf)
    return g, lsum[0, 0] / (_ROWS * _D)


def _phase_weights(dec_w):
    t = jnp.array([[[1, 0, 0], [0, 1, 1]],
                   [[1, 1, 0], [0, 0, 1]]], jnp.float32)
    w2 = jnp.einsum('aup,bvq,oipq->abuvio', t, t, dec_w)
    co, ci = dec_w.shape[0], dec_w.shape[1]
    return w2.reshape(4, 4, ci, co)


def _slice_weights(w2f):
    ci, co = w2f.shape[2], w2f.shape[3]
    blocks = []
    for r in (1, 2, 3):
        for cc in (0, 1, 2):
            cols = []
            for a in (0, 1):
                for b in (0, 1):
                    u, v = r - 1 - a, cc - b
                    if 0 <= u <= 1 and 0 <= v <= 1:
                        cols.append(w2f[2 * a + b, 2 * u + v])
                    else:
                        cols.append(jnp.zeros((ci, co), jnp.float32))
            blocks.append(jnp.concatenate(cols, axis=1))
    return jnp.stack(blocks).astype(jnp.bfloat16)


def _make_upconv_kernel(H, Wp, relu, out_dtype, nchunks, pack_phases):
    L = H * Wp
    CH = L // nchunks
    LEN = (H + 4) * Wp

    def _k(x_ref, w_ref, b_ref, o_ref, sc_ref):
        for ci, c in enumerate((-1, 0, 1)):
            sc_ref[ci, pl.ds(0, LEN - 16), :] = x_ref[0, pl.ds(8 + c, LEN - 16), :]
        for ch in range(nchunks):
            if pack_phases:
                acc = jnp.zeros((CH, w_ref.shape[2]), jnp.float32)
                for r in (1, 2, 3):
                    for cc in (0, 1, 2):
                        xs = sc_ref[cc, pl.ds(r * Wp - 8 + ch * CH, CH), :]
                        acc += jnp.dot(xs, w_ref[3 * (r - 1) + cc],
                                       preferred_element_type=jnp.float32)
                acc = acc + b_ref[...]
                if relu:
                    acc = jnp.maximum(acc, 0.0)
                o_ref[0, pl.ds(ch * CH, CH), :] = acc.astype(out_dtype)
            else:
                for a in (0, 1):
                    for b in (0, 1):
                        acc = jnp.zeros((CH, w_ref.shape[3]), jnp.float32)
                        for u in (0, 1):
                            for v in (0, 1):
                                xs = sc_ref[b + v, pl.ds(
                                    (1 + a + u) * Wp - 8 + ch * CH, CH), :]
                                acc += jnp.dot(
                                    xs, w_ref[2 * a + b, 2 * u + v],
                                    preferred_element_type=jnp.float32)
                        acc = acc + b_ref[...]
                        if relu:
                            acc = jnp.maximum(acc, 0.0)
                        o_ref[0, 2 * a + b, pl.ds(ch * CH, CH), :] = (
                            acc.astype(out_dtype))

    return _k


def _upconv(g, w, bias, relu, out_dtype):
    n, h, wdim, ci = g.shape
    co = w.shape[0]
    wp = ((wdim + 2 + 7) // 8) * 8
    w2f = _phase_weights(w)
    gp = jnp.pad(g.astype(jnp.bfloat16),
                 ((0, 0), (2, 2), (1, wp - wdim - 1), (0, 0)))
    gp = gp.reshape(n, (h + 4) * wp, ci)
    l = h * wp
    nchunks = 1 if l <= 4096 else 4
    pack = co < 128
    if pack:
        wk = _slice_weights(w2f)
        bk = jnp.tile(bias[None, :], (1, 4)).astype(jnp.float32)
        w_spec = pl.BlockSpec((9, ci, 4 * co), lambda i: (0, 0, 0))
        b_spec = pl.BlockSpec((1, 4 * co), lambda i: (0, 0))
        out_specs = pl.BlockSpec((1, l, 4 * co), lambda i: (i, 0, 0))
        out_shape = jax.ShapeDtypeStruct((n, l, 4 * co), out_dtype)
    else:
        wk = w2f.astype(jnp.bfloat16)
        bk = bias[None, :].astype(jnp.float32)
        w_spec = pl.BlockSpec((4, 4, ci, co), lambda i: (0, 0, 0, 0))
        b_spec = pl.BlockSpec((1, co), lambda i: (0, 0))
        out_specs = pl.BlockSpec((1, 4, l, co), lambda i: (i, 0, 0, 0))
        out_shape = jax.ShapeDtypeStruct((n, 4, l, co), out_dtype)
    out = pl.pallas_call(
        _make_upconv_kernel(h, wp, relu, out_dtype, nchunks, pack),
        grid=(n,),
        in_specs=[
            pl.BlockSpec((1, (h + 4) * wp, ci), lambda i: (i, 0, 0)),
            w_spec,
            b_spec,
        ],
        out_specs=out_specs,
        out_shape=out_shape,
        scratch_shapes=[pltpu.VMEM((3, (h + 4) * wp - 16, ci), jnp.bfloat16)],
    )(gp, wk, bk)
    if pack:
        out = out.reshape(n, h, wp, 2, 2, co)[:, :, 1:wdim + 1]
        out = jnp.transpose(out, (0, 1, 3, 2, 4, 5))
    else:
        out = out.reshape(n, 2, 2, h, wp, co)[:, :, :, :, 1:wdim + 1, :]
        out = jnp.transpose(out, (0, 3, 1, 4, 2, 5))
    return out.reshape(n, 2 * h, 2 * wdim, co)


def kernel(x, enc_w1, enc_b1, enc_w2, enc_b2, enc_w3, enc_b3, quant_w,
           quant_b, codebook, pq_w, pq_b, dec_w1, dec_b1, dec_w2, dec_b2,
           dec_w3, dec_b3):
    h = jax.nn.relu(_conv(x, enc_w1, enc_b1, stride=2))
    h = jax.nn.relu(_conv(h, enc_w2, enc_b2, stride=2))
    h = _conv(h, enc_w3, enc_b3, stride=2)

    z = _conv(h, quant_w, quant_b, stride=1, pad=0)
    z = jnp.transpose(z, (0, 2, 3, 1))
    zf = z.reshape(-1, 64)
    dd = (jnp.sum(zf * zf, axis=1, keepdims=True) - 2.0 * (zf @ codebook.T)
          + jnp.sum(codebook * codebook, axis=1)[None, :])
    idx = jnp.argmin(dd, axis=1).astype(jnp.int32)

    pqm = pq_w[:, :, 0, 0].T
    cb2 = _pq_table(codebook, pqm, pq_b)
    g_vec, commit_loss = _vq_post(idx, codebook, cb2, zf)

    g = g_vec.reshape(4, 28, 28, 256)
    g = _upconv(g, dec_w1, dec_b1, relu=True, out_dtype=jnp.bfloat16)
    g = _upconv(g, dec_w2, dec_b2, relu=True, out_dtype=jnp.bfloat16)
    g = _upconv(g, dec_w3, dec_b3, relu=False, out_dtype=jnp.float32)
    decoded = jnp.transpose(g, (0, 3, 1, 2))
    return (commit_loss, decoded)


# speedup vs baseline: 1.4389x; 1.0324x over previous
import jax
import jax.numpy as jnp
import numpy as np
from jax.experimental import pallas as pl
from jax.experimental.pallas import tpu as pltpu


def _conv(x, w, b, stride=1, pad=1):
    y = jax.lax.conv_general_dilated(
        x, w, (stride, stride), [(pad, pad), (pad, pad)],
        dimension_numbers=('NCHW', 'OIHW', 'NCHW'))
    return y + b[None, :, None, None]


def _up2(x):
    return jnp.repeat(jnp.repeat(x, 2, axis=2), 2, axis=3)


_ROWS = 3136
_BLK = 448
_K = 1024
_D = 64


def _pq_table_kernel(cb_ref, w_ref, b_ref, o_ref):
    o_ref[...] = (jnp.dot(cb_ref[...], w_ref[...],
                          preferred_element_type=jnp.float32)
                  + b_ref[...]).astype(jnp.bfloat16)


def _pq_table(cb, pqm, pqb):
    return pl.pallas_call(
        _pq_table_kernel,
        out_shape=jax.ShapeDtypeStruct((_K, 256), jnp.bfloat16),
    )(cb, pqm, pqb[None, :])


def _vq_post_kernel(idx_ref, cb_ref, cb2_ref, zf_ref, g_ref, ls_ref):
    i = pl.program_id(0)
    oh = (jax.lax.broadcasted_iota(jnp.int32, (_BLK, _K), 1)
          == idx_ref[...]).astype(jnp.float32)
    q = jnp.dot(oh, cb_ref[...], preferred_element_type=jnp.float32)
    dq = q - zf_ref[...]
    part = jnp.sum(dq * dq).reshape(1, 1)

    @pl.when(i == 0)
    def _init():
        ls_ref[...] = jnp.zeros_like(part)

    ls_ref[...] += part
    g_ref[...] = jnp.dot(oh.astype(jnp.bfloat16), cb2_ref[...],
                         preferred_element_type=jnp.float32
                         ).astype(jnp.bfloat16)


def _vq_post(idx, cb, cb2, zf):
    g, lsum = pl.pallas_call(
        _vq_post_kernel,
        grid=(_ROWS // _BLK,),
        in_specs=[
            pl.BlockSpec((_BLK, 1), lambda i: (i, 0)),
            pl.BlockSpec((_K, _D), lambda i: (0, 0)),
            pl.BlockSpec((_K, 256), lambda i: (0, 0)),
            pl.BlockSpec((_BLK, _D), lambda i: (i, 0)),
        ],
        out_specs=[
            pl.BlockSpec((_BLK, 256), lambda i: (i, 0)),
            pl.BlockSpec((1, 1), lambda i: (0, 0)),
        ],
        out_shape=[
            jax.ShapeDtypeStruct((_ROWS, 256), jnp.bfloat16),
            jax.ShapeDtypeStruct((1, 1), jnp.float32),
        ],
    )(idx[:, None], cb, cb2, zf)
    return g, lsum[0, 0] / (_ROWS * _D)


def _phase_weights(dec_w):
    t = jnp.array([[[1, 0, 0], [0, 1, 1]],
                   [[1, 1, 0], [0, 0, 1]]], jnp.float32)
    w2 = jnp.einsum('aup,bvq,oipq->abuvio', t, t, dec_w)
    co, ci = dec_w.shape[0], dec_w.shape[1]
    return w2.reshape(4, 4, ci, co)


def _slice_weights(w2f):
    ci, co = w2f.shape[2], w2f.shape[3]
    blocks = []
    for r in (1, 2, 3):
        for cc in (0, 1, 2):
            cols = []
            for a in (0, 1):
                for b in (0, 1):
                    u, v = r - 1 - a, cc - b
                    if 0 <= u <= 1 and 0 <= v <= 1:
                        cols.append(w2f[2 * a + b, 2 * u + v])
                    else:
                        cols.append(jnp.zeros((ci, co), jnp.float32))
            blocks.append(jnp.concatenate(cols, axis=1))
    return jnp.stack(blocks).astype(jnp.bfloat16)


def _make_upconv_kernel(H, Wp, relu, out_dtype, nchunks, pack_phases,
                        phase_in, ci_dim):
    L = H * Wp
    CH = L // nchunks
    LEN = (H + 4) * Wp

    def _stage_dense(x_ref, sc_ref):
        for k, c in enumerate((-1, 0, 1)):
            sc_ref[k, pl.ds(0, LEN - 16), :] = x_ref[0, pl.ds(8 + c, LEN - 16), :]

    def _stage_phases(x_ref, sc_ref):
        h_in, wp_in, w_in = phase_in
        z = jnp.zeros((h_in, 1, ci_dim), x_ref.dtype)
        vs = []
        for a in (0, 1):
            v0 = x_ref[0, 2 * a + 0].reshape(h_in, wp_in, ci_dim)
            v1 = x_ref[0, 2 * a + 1].reshape(h_in, wp_in, ci_dim)
            v0s = jnp.concatenate([v0[:, 1:, :], z], axis=1)
            vs.append(jnp.stack([v1, v0s], axis=2).reshape(h_in, Wp, ci_dim))
        xv = jnp.stack(vs, axis=1).reshape(H, Wp, ci_dim)
        colid = jax.lax.broadcasted_iota(jnp.int32, (H, Wp, ci_dim), 1)
        xv = jnp.where((colid >= 1) & (colid <= 2 * w_in), xv,
                       jnp.zeros_like(xv))
        zr = jnp.zeros((2 * Wp, ci_dim), xv.dtype)
        xp = jnp.concatenate([zr, xv.reshape(L, ci_dim), zr], axis=0)
        for k in (0, 1, 2):
            sc_ref[k, pl.ds(0, LEN - 16), :] = xp[7 + k:7 + k + LEN - 16, :]

    def _tap(sc_ref, r, cc, ch):
        return sc_ref[cc, pl.ds(r * Wp - 8 + ch * CH, CH), :]

    def _k(x_ref, w_ref, b_ref, o_ref, sc_ref):
        if phase_in is None:
            _stage_dense(x_ref, sc_ref)
        else:
            _stage_phases(x_ref, sc_ref)
        for ch in range(nchunks):
            if pack_phases:
                acc = jnp.zeros((CH, w_ref.shape[2]), jnp.float32)
                for r in (1, 2, 3):
                    for cc in (0, 1, 2):
                        xs = _tap(sc_ref, r, cc, ch)
                        acc += jnp.dot(xs, w_ref[3 * (r - 1) + cc],
                                       preferred_element_type=jnp.float32)
                acc = acc + b_ref[...]
                if relu:
                    acc = jnp.maximum(acc, 0.0)
                o_ref[0, pl.ds(ch * CH, CH), :] = acc.astype(out_dtype)
            else:
                for a in (0, 1):
                    for b in (0, 1):
                        acc = jnp.zeros((CH, w_ref.shape[3]), jnp.float32)
                        for u in (0, 1):
                            for v in (0, 1):
                                xs = _tap(sc_ref, 1 + a + u, b + v, ch)
                                acc += jnp.dot(
                                    xs, w_ref[2 * a + b, 2 * u + v],
                                    preferred_element_type=jnp.float32)
                        acc = acc + b_ref[...]
                        if relu:
                            acc = jnp.maximum(acc, 0.0)
                        o_ref[0, 2 * a + b, pl.ds(ch * CH, CH), :] = (
                            acc.astype(out_dtype))

    return _k


def _upconv(g, w, bias, relu, out_dtype, phase_in=None, raw_out=False):
    co = w.shape[0]
    if phase_in is None:
        n, h, wdim, ci = g.shape
        wp = ((wdim + 2 + 7) // 8) * 8
        gp = jnp.pad(g.astype(jnp.bfloat16),
                     ((0, 0), (2, 2), (1, wp - wdim - 1), (0, 0)))
        gp = gp.reshape(n, (h + 4) * wp, ci)
        x_spec = pl.BlockSpec((1, (h + 4) * wp, ci), lambda i: (i, 0, 0))
    else:
        h_in, wp_in, w_in = phase_in
        n, ci = g.shape[0], g.shape[3]
        h, wdim, wp = 2 * h_in, 2 * w_in, 2 * wp_in
        gp = g
        x_spec = pl.BlockSpec((1, 4, h_in * wp_in, ci),
                              lambda i: (i, 0, 0, 0))
    w2f = _phase_weights(w)
    l = h * wp
    nchunks = 1 if l <= 4096 else 4
    pack = co < 128
    if pack:
        wk = _slice_weights(w2f)
        bk = jnp.tile(bias[None, :], (1, 4)).astype(jnp.float32)
        w_spec = pl.BlockSpec((9, ci, 4 * co), lambda i: (0, 0, 0))
        b_spec = pl.BlockSpec((1, 4 * co), lambda i: (0, 0))
        out_specs = pl.BlockSpec((1, l, 4 * co), lambda i: (i, 0, 0))
        out_shape = jax.ShapeDtypeStruct((n, l, 4 * co), out_dtype)
    else:
        wk = w2f.astype(jnp.bfloat16)
        bk = bias[None, :].astype(jnp.float32)
        w_spec = pl.BlockSpec((4, 4, ci, co), lambda i: (0, 0, 0, 0))
        b_spec = pl.BlockSpec((1, co), lambda i: (0, 0))
        out_specs = pl.BlockSpec((1, 4, l, co), lambda i: (i, 0, 0, 0))
        out_shape = jax.ShapeDtypeStruct((n, 4, l, co), out_dtype)
    scratch = pltpu.VMEM((3, (h + 4) * wp - 16, ci), jnp.bfloat16)
    out = pl.pallas_call(
        _make_upconv_kernel(h, wp, relu, out_dtype, nchunks, pack,
                            phase_in, ci),
        grid=(n,),
        in_specs=[x_spec, w_spec, b_spec],
        out_specs=out_specs,
        out_shape=out_shape,
        scratch_shapes=[scratch],
    )(gp, wk, bk)
    if raw_out and not pack:
        return out
    if pack:
        out = out.reshape(n, h, wp, 2, 2, co)[:, :, 1:wdim + 1]
        out = jnp.transpose(out, (0, 1, 3, 2, 4, 5))
    else:
        out = out.reshape(n, 2, 2, h, wp, co)[:, :, :, :, 1:wdim + 1, :]
        out = jnp.transpose(out, (0, 3, 1, 4, 2, 5))
    return out.reshape(n, 2 * h, 2 * wdim, co)


def kernel(x, enc_w1, enc_b1, enc_w2, enc_b2, enc_w3, enc_b3, quant_w,
           quant_b, codebook, pq_w, pq_b, dec_w1, dec_b1, dec_w2, dec_b2,
           dec_w3, dec_b3):
    h = jax.nn.relu(_conv(x, enc_w1, enc_b1, stride=2))
    h = jax.nn.relu(_conv(h, enc_w2, enc_b2, stride=2))
    h = _conv(h, enc_w3, enc_b3, stride=2)

    z = _conv(h, quant_w, quant_b, stride=1, pad=0)
    z = jnp.transpose(z, (0, 2, 3, 1))
    zf = z.reshape(-1, 64)
    dd = (jnp.sum(zf * zf, axis=1, keepdims=True) - 2.0 * (zf @ codebook.T)
          + jnp.sum(codebook * codebook, axis=1)[None, :])
    idx = jnp.argmin(dd, axis=1).astype(jnp.int32)

    pqm = pq_w[:, :, 0, 0].T
    cb2 = _pq_table(codebook, pqm, pq_b)
    g_vec, commit_loss = _vq_post(idx, codebook, cb2, zf)

    g = g_vec.reshape(4, 28, 28, 256)
    p1 = _upconv(g, dec_w1, dec_b1, relu=True, out_dtype=jnp.bfloat16,
                 raw_out=True)
    p2 = _upconv(p1, dec_w2, dec_b2, relu=True, out_dtype=jnp.bfloat16,
                 phase_in=(28, 32, 28), raw_out=True)
    g = _upconv(p2, dec_w3, dec_b3, relu=False, out_dtype=jnp.float32,
                phase_in=(56, 64, 56))
    decoded = jnp.transpose(g, (0, 3, 1, 2))
    return (commit_loss, decoded)


# speedup vs baseline: 1.4559x; 1.0119x over previous
import jax
import jax.numpy as jnp
import numpy as np
from jax.experimental import pallas as pl
from jax.experimental.pallas import tpu as pltpu


def _conv(x, w, b, stride=1, pad=1):
    y = jax.lax.conv_general_dilated(
        x, w, (stride, stride), [(pad, pad), (pad, pad)],
        dimension_numbers=('NCHW', 'OIHW', 'NCHW'))
    return y + b[None, :, None, None]


def _up2(x):
    return jnp.repeat(jnp.repeat(x, 2, axis=2), 2, axis=3)


_ROWS = 3136
_BLK = 448
_K = 1024
_D = 64


def _pq_table_kernel(cb_ref, w_ref, b_ref, o_ref):
    o_ref[...] = (jnp.dot(cb_ref[...], w_ref[...],
                          preferred_element_type=jnp.float32)
                  + b_ref[...]).astype(jnp.bfloat16)


def _pq_table(cb, pqm, pqb):
    return pl.pallas_call(
        _pq_table_kernel,
        out_shape=jax.ShapeDtypeStruct((_K, 256), jnp.bfloat16),
    )(cb, pqm, pqb[None, :])


def _vq_post_kernel(idx_ref, cb_ref, cb2_ref, zf_ref, g_ref, ls_ref):
    i = pl.program_id(0)
    oh = (jax.lax.broadcasted_iota(jnp.int32, (_BLK, _K), 1)
          == idx_ref[...]).astype(jnp.float32)
    q = jnp.dot(oh, cb_ref[...], preferred_element_type=jnp.float32)
    dq = q - zf_ref[...]
    part = jnp.sum(dq * dq).reshape(1, 1)

    @pl.when(i == 0)
    def _init():
        ls_ref[...] = jnp.zeros_like(part)

    ls_ref[...] += part
    g_ref[...] = jnp.dot(oh.astype(jnp.bfloat16), cb2_ref[...],
                         preferred_element_type=jnp.float32
                         ).astype(jnp.bfloat16)


def _vq_post(idx, cb, cb2, zf):
    g, lsum = pl.pallas_call(
        _vq_post_kernel,
        grid=(_ROWS // _BLK,),
        in_specs=[
            pl.BlockSpec((_BLK, 1), lambda i: (i, 0)),
            pl.BlockSpec((_K, _D), lambda i: (0, 0)),
            pl.BlockSpec((_K, 256), lambda i: (0, 0)),
            pl.BlockSpec((_BLK, _D), lambda i: (i, 0)),
        ],
        out_specs=[
            pl.BlockSpec((_BLK, 256), lambda i: (i, 0)),
            pl.BlockSpec((1, 1), lambda i: (0, 0)),
        ],
        out_shape=[
            jax.ShapeDtypeStruct((_ROWS, 256), jnp.bfloat16),
            jax.ShapeDtypeStruct((1, 1), jnp.float32),
        ],
    )(idx[:, None], cb, cb2, zf)
    return g, lsum[0, 0] / (_ROWS * _D)


def _phase_weights(dec_w):
    t = jnp.array([[[1, 0, 0], [0, 1, 1]],
                   [[1, 1, 0], [0, 0, 1]]], jnp.float32)
    w2 = jnp.einsum('aup,bvq,oipq->abuvio', t, t, dec_w)
    co, ci = dec_w.shape[0], dec_w.shape[1]
    return w2.reshape(4, 4, ci, co)


def _slice_weights(w2f):
    ci, co = w2f.shape[2], w2f.shape[3]
    blocks = []
    for r in (1, 2, 3):
        for cc in (0, 1, 2):
            cols = []
            for a in (0, 1):
                for b in (0, 1):
                    u, v = r - 1 - a, cc - b
                    if 0 <= u <= 1 and 0 <= v <= 1:
                        cols.append(w2f[2 * a + b, 2 * u + v])
                    else:
                        cols.append(jnp.zeros((ci, co), jnp.float32))
            blocks.append(jnp.concatenate(cols, axis=1))
    return jnp.stack(blocks).astype(jnp.bfloat16)


def _make_upconv_kernel(H, Wp, relu, out_dtype, nchunks, pack_phases,
                        phase_in, ci_dim, wdim):
    L = H * Wp
    CH = L // nchunks
    LEN = (H + 4) * Wp

    def _finish(xv, sc_ref):
        zr = jnp.zeros((2 * Wp, ci_dim), xv.dtype)
        xp = jnp.concatenate([zr, xv.reshape(L, ci_dim), zr], axis=0)
        for k in (0, 1, 2):
            sc_ref[k, pl.ds(0, LEN - 16), :] = xp[7 + k:7 + k + LEN - 16, :]

    def _stage_dense(x_ref, sc_ref):
        xr = x_ref[0].reshape(H, wdim, ci_dim)
        xv = jnp.concatenate(
            [jnp.zeros((H, 1, ci_dim), xr.dtype), xr,
             jnp.zeros((H, Wp - wdim - 1, ci_dim), xr.dtype)], axis=1)
        _finish(xv, sc_ref)

    def _stage_phases(x_ref, sc_ref):
        h_in, wp_in, w_in = phase_in
        z = jnp.zeros((h_in, 1, ci_dim), x_ref.dtype)
        vs = []
        for a in (0, 1):
            v0 = x_ref[0, 2 * a + 0].reshape(h_in, wp_in, ci_dim)
            v1 = x_ref[0, 2 * a + 1].reshape(h_in, wp_in, ci_dim)
            v0s = jnp.concatenate([v0[:, 1:, :], z], axis=1)
            vs.append(jnp.stack([v1, v0s], axis=2).reshape(h_in, Wp, ci_dim))
        xv = jnp.stack(vs, axis=1).reshape(H, Wp, ci_dim)
        colid = jax.lax.broadcasted_iota(jnp.int32, (H, Wp, ci_dim), 1)
        xv = jnp.where((colid >= 1) & (colid <= 2 * w_in), xv,
                       jnp.zeros_like(xv))
        _finish(xv, sc_ref)

    def _tap(sc_ref, r, cc, ch):
        return sc_ref[cc, pl.ds(r * Wp - 8 + ch * CH, CH), :]

    def _k(x_ref, w_ref, b_ref, o_ref, sc_ref):
        if phase_in is None:
            _stage_dense(x_ref, sc_ref)
        else:
            _stage_phases(x_ref, sc_ref)
        for ch in range(nchunks):
            if pack_phases:
                acc = jnp.zeros((CH, w_ref.shape[2]), jnp.float32)
                for r in (1, 2, 3):
                    for cc in (0, 1, 2):
                        xs = _tap(sc_ref, r, cc, ch)
                        acc += jnp.dot(xs, w_ref[3 * (r - 1) + cc],
                                       preferred_element_type=jnp.float32)
                acc = acc + b_ref[...]
                if relu:
                    acc = jnp.maximum(acc, 0.0)
                o_ref[0, pl.ds(ch * CH, CH), :] = acc.astype(out_dtype)
            else:
                for a in (0, 1):
                    for b in (0, 1):
                        acc = jnp.zeros((CH, w_ref.shape[3]), jnp.float32)
                        for u in (0, 1):
                            for v in (0, 1):
                                xs = _tap(sc_ref, 1 + a + u, b + v, ch)
                                acc += jnp.dot(
                                    xs, w_ref[2 * a + b, 2 * u + v],
                                    preferred_element_type=jnp.float32)
                        acc = acc + b_ref[...]
                        if relu:
                            acc = jnp.maximum(acc, 0.0)
                        o_ref[0, 2 * a + b, pl.ds(ch * CH, CH), :] = (
                            acc.astype(out_dtype))

    return _k


def _upconv(g, w, bias, relu, out_dtype, phase_in=None, raw_out=False):
    co = w.shape[0]
    if phase_in is None:
        n, h, wdim, ci = g.shape
        wp = ((wdim + 2 + 7) // 8) * 8
        gp = g.astype(jnp.bfloat16).reshape(n, h * wdim, ci)
        x_spec = pl.BlockSpec((1, h * wdim, ci), lambda i: (i, 0, 0))
    else:
        h_in, wp_in, w_in = phase_in
        n, ci = g.shape[0], g.shape[3]
        h, wdim, wp = 2 * h_in, 2 * w_in, 2 * wp_in
        gp = g
        x_spec = pl.BlockSpec((1, 4, h_in * wp_in, ci),
                              lambda i: (i, 0, 0, 0))
    w2f = _phase_weights(w)
    l = h * wp
    nchunks = 1 if l <= 4096 else 4
    pack = co < 128
    if pack:
        wk = _slice_weights(w2f)
        bk = jnp.tile(bias[None, :], (1, 4)).astype(jnp.float32)
        w_spec = pl.BlockSpec((9, ci, 4 * co), lambda i: (0, 0, 0))
        b_spec = pl.BlockSpec((1, 4 * co), lambda i: (0, 0))
        out_specs = pl.BlockSpec((1, l, 4 * co), lambda i: (i, 0, 0))
        out_shape = jax.ShapeDtypeStruct((n, l, 4 * co), out_dtype)
    else:
        wk = w2f.astype(jnp.bfloat16)
        bk = bias[None, :].astype(jnp.float32)
        w_spec = pl.BlockSpec((4, 4, ci, co), lambda i: (0, 0, 0, 0))
        b_spec = pl.BlockSpec((1, co), lambda i: (0, 0))
        out_specs = pl.BlockSpec((1, 4, l, co), lambda i: (i, 0, 0, 0))
        out_shape = jax.ShapeDtypeStruct((n, 4, l, co), out_dtype)
    scratch = pltpu.VMEM((3, (h + 4) * wp - 16, ci), jnp.bfloat16)
    out = pl.pallas_call(
        _make_upconv_kernel(h, wp, relu, out_dtype, nchunks, pack,
                            phase_in, ci, wdim),
        grid=(n,),
        in_specs=[x_spec, w_spec, b_spec],
        out_specs=out_specs,
        out_shape=out_shape,
        scratch_shapes=[scratch],
    )(gp, wk, bk)
    if raw_out and not pack:
        return out
    if pack:
        out = out.reshape(n, h, wp, 2, 2, co)[:, :, 1:wdim + 1]
        out = jnp.transpose(out, (0, 1, 3, 2, 4, 5))
    else:
        out = out.reshape(n, 2, 2, h, wp, co)[:, :, :, :, 1:wdim + 1, :]
        out = jnp.transpose(out, (0, 3, 1, 4, 2, 5))
    return out.reshape(n, 2 * h, 2 * wdim, co)


def kernel(x, enc_w1, enc_b1, enc_w2, enc_b2, enc_w3, enc_b3, quant_w,
           quant_b, codebook, pq_w, pq_b, dec_w1, dec_b1, dec_w2, dec_b2,
           dec_w3, dec_b3):
    h = jax.nn.relu(_conv(x, enc_w1, enc_b1, stride=2))
    h = jax.nn.relu(_conv(h, enc_w2, enc_b2, stride=2))
    h = _conv(h, enc_w3, enc_b3, stride=2)

    z = _conv(h, quant_w, quant_b, stride=1, pad=0)
    z = jnp.transpose(z, (0, 2, 3, 1))
    zf = z.reshape(-1, 64)
    dd = (jnp.sum(zf * zf, axis=1, keepdims=True) - 2.0 * (zf @ codebook.T)
          + jnp.sum(codebook * codebook, axis=1)[None, :])
    idx = jnp.argmin(dd, axis=1).astype(jnp.int32)

    pqm = pq_w[:, :, 0, 0].T
    cb2 = _pq_table(codebook, pqm, pq_b)
    g_vec, commit_loss = _vq_post(idx, codebook, cb2, zf)

    g = g_vec.reshape(4, 28, 28, 256)
    p1 = _upconv(g, dec_w1, dec_b1, relu=True, out_dtype=jnp.bfloat16,
                 raw_out=True)
    p2 = _upconv(p1, dec_w2, dec_b2, relu=True, out_dtype=jnp.bfloat16,
                 phase_in=(28, 32, 28), raw_out=True)
    g = _upconv(p2, dec_w3, dec_b3, relu=False, out_dtype=jnp.float32,
                phase_in=(56, 64, 56))
    decoded = jnp.transpose(g, (0, 3, 1, 2))
    return (commit_loss, decoded)
